# Initial kernel scaffold; baseline (speedup 1.0000x reference)
import jax
import jax.numpy as jnp
from jax.experimental import pallas as pl


def kernel(x, edge_index, W1, b1, g1, beta1, W2, b2, g2, beta2):
    raise NotImplementedError("write your pallas kernel here")


# speedup vs baseline: 13.7320x; 13.7320x over previous
import functools

import jax
import jax.numpy as jnp
from jax import lax
from jax.experimental import pallas as pl
from jax.experimental.pallas import tpu as pltpu
from jax.experimental.pallas import tpu_sc as plsc

_N = 10000
_E = 320000
_D = 128
_NC = 2
_NS = 16
_NW = _NC * _NS
_EPT = _E // _NW
_CHUNK = 128
_NFULL = _EPT // _CHUNK
_TAIL = _EPT - _NFULL * _CHUNK
_NPAD = 10240
_STRIPE = _NPAD // _NS

_mesh = plsc.VectorSubcoreMesh(core_axis_name="core", subcore_axis_name="subcore")


def _deg_body(dst_hbm, deg_hbm, idx_v, idx_t, ones_v, acc):
    c = lax.axis_index("core")
    s = lax.axis_index("subcore")
    wid = c * _NS + s

    @pl.loop(0, _CHUNK)
    def _(r):
        @pl.loop(0, _D // 16)
        def _(k):
            ones_v[r, pl.ds(k * 16, 16)] = jnp.zeros((16,), jnp.float32)

    @pl.loop(0, _STRIPE // _CHUNK)
    def _(j):
        pltpu.sync_copy(ones_v, acc.at[pl.ds(s * _STRIPE + j * _CHUNK, _CHUNK)])
    plsc.subcore_barrier()

    @pl.loop(0, _CHUNK)
    def _(r):
        ones_v[r, pl.ds(0, 16)] = jnp.ones((16,), jnp.float32)

    base = wid * _EPT

    @pl.loop(0, _NFULL)
    def _(i):
        pltpu.sync_copy(dst_hbm.at[pl.ds(base + i * _CHUNK, _CHUNK)], idx_v)
        pltpu.sync_copy(ones_v, acc.at[idx_v], add=True)

    pltpu.sync_copy(dst_hbm.at[pl.ds(base + _NFULL * _CHUNK, _TAIL)], idx_t)
    pltpu.sync_copy(ones_v.at[pl.ds(0, _TAIL)], acc.at[idx_t], add=True)

    plsc.subcore_barrier()

    @pl.loop(0, _STRIPE // _CHUNK)
    def _(j):
        r0 = s * _STRIPE + j * _CHUNK
        pltpu.sync_copy(acc.at[pl.ds(r0, _CHUNK)], deg_hbm.at[c].at[pl.ds(r0, _CHUNK)])


@jax.jit
def _deg_call(dst):
    f = pl.kernel(
        _deg_body,
        out_type=jax.ShapeDtypeStruct((_NC, _NPAD, _D), jnp.float32),
        mesh=_mesh,
        scratch_types=[
            pltpu.VMEM((_CHUNK,), jnp.int32),
            pltpu.VMEM((_TAIL,), jnp.int32),
            pltpu.VMEM((_CHUNK, _D), jnp.float32),
            pltpu.VMEM_SHARED((_NPAD, _D), jnp.float32),
        ],
    )
    return f(dst)


def _agg_body(hs_hbm, src_hbm, dst_hbm, out_hbm, idx_s, idx_d, idx_s16, idx_d16,
              rows, acc):
    c = lax.axis_index("core")
    s = lax.axis_index("subcore")
    wid = c * _NS + s

    @pl.loop(0, _CHUNK)
    def _(r):
        @pl.loop(0, _D // 16)
        def _(k):
            rows[r, pl.ds(k * 16, 16)] = jnp.zeros((16,), jnp.float32)

    @pl.loop(0, _STRIPE // _CHUNK)
    def _(j):
        pltpu.sync_copy(rows, acc.at[pl.ds(s * _STRIPE + j * _CHUNK, _CHUNK)])
    plsc.subcore_barrier()

    base = wid * _EPT

    @pl.loop(0, _NFULL)
    def _(i):
        e0 = base + i * _CHUNK
        pltpu.sync_copy(src_hbm.at[pl.ds(e0, _CHUNK)], idx_s)
        pltpu.sync_copy(dst_hbm.at[pl.ds(e0, _CHUNK)], idx_d)
        pltpu.sync_copy(hs_hbm.at[idx_s], rows)
        pltpu.sync_copy(rows, acc.at[idx_d], add=True)

    t0 = base + _NFULL * _CHUNK
    pltpu.sync_copy(src_hbm.at[pl.ds(t0, _TAIL)], idx_s16)
    pltpu.sync_copy(dst_hbm.at[pl.ds(t0, _TAIL)], idx_d16)
    pltpu.sync_copy(hs_hbm.at[idx_s16], rows.at[pl.ds(0, _TAIL)])
    pltpu.sync_copy(rows.at[pl.ds(0, _TAIL)], acc.at[idx_d16], add=True)

    plsc.subcore_barrier()

    @pl.loop(0, _STRIPE // _CHUNK)
    def _(j):
        r0 = s * _STRIPE + j * _CHUNK
        pltpu.sync_copy(acc.at[pl.ds(r0, _CHUNK)], out_hbm.at[c].at[pl.ds(r0, _CHUNK)])


@jax.jit
def _agg_call(hs, src, dst):
    f = pl.kernel(
        _agg_body,
        out_type=jax.ShapeDtypeStruct((_NC, _NPAD, _D), jnp.float32),
        mesh=_mesh,
        scratch_types=[
            pltpu.VMEM((_CHUNK,), jnp.int32),
            pltpu.VMEM((_CHUNK,), jnp.int32),
            pltpu.VMEM((_TAIL,), jnp.int32),
            pltpu.VMEM((_TAIL,), jnp.int32),
            pltpu.VMEM((_CHUNK, _D), jnp.float32),
            pltpu.VMEM_SHARED((_NPAD, _D), jnp.float32),
        ],
    )
    return f(hs, src, dst)


_BN = 2000
_GRID = _N // _BN


def _row_spec():
    return pl.BlockSpec((_BN, _D), lambda i: (i, 0))


def _col_spec():
    return pl.BlockSpec((_BN, 1), lambda i: (i, 0))


def _full_spec():
    return pl.BlockSpec((_D, _D), lambda i: (0, 0))


def _vec_spec():
    return pl.BlockSpec((1, _D), lambda i: (0, 0))


def _stage1_body(x_ref, w_ref, da_ref, db_ref, h_ref, hs_ref, dinv_ref):
    deg = da_ref[...] + db_ref[...] + 1.0
    dinv = lax.rsqrt(deg)
    h = jnp.dot(x_ref[...], w_ref[...], preferred_element_type=jnp.float32)
    h_ref[...] = h
    hs_ref[...] = h * dinv
    dinv_ref[...] = dinv


@jax.jit
def _stage1(x, W1, da, db):
    return pl.pallas_call(
        _stage1_body,
        grid=(_GRID,),
        in_specs=[_row_spec(), _full_spec(), _col_spec(), _col_spec()],
        out_specs=[_row_spec(), _row_spec(), _col_spec()],
        out_shape=[
            jax.ShapeDtypeStruct((_N, _D), jnp.float32),
            jax.ShapeDtypeStruct((_N, _D), jnp.float32),
            jax.ShapeDtypeStruct((_N, 1), jnp.float32),
        ],
    )(x, W1, da, db)


def _layer_norm(agg, g, beta):
    mu = jnp.mean(agg, axis=-1, keepdims=True)
    var = jnp.mean((agg - mu) ** 2, axis=-1, keepdims=True)
    return (agg - mu) * lax.rsqrt(var + 1e-5) * g + beta


def _stage2_body(sa_ref, sb_ref, h1_ref, dinv_ref, g_ref, beta_ref, b_ref,
                 w_ref, h2_ref, hs2_ref):
    dinv = dinv_ref[...]
    agg = dinv * (sa_ref[...] + sb_ref[...]) + dinv * dinv * h1_ref[...] + b_ref[...]
    y = jnp.maximum(_layer_norm(agg, g_ref[...], beta_ref[...]), 0.0)
    h2 = jnp.dot(y, w_ref[...], preferred_element_type=jnp.float32)
    h2_ref[...] = h2
    hs2_ref[...] = h2 * dinv


@jax.jit
def _stage2(sa, sb, h1, dinv, g1, beta1, b1, W2):
    return pl.pallas_call(
        _stage2_body,
        grid=(_GRID,),
        in_specs=[_row_spec(), _row_spec(), _row_spec(), _col_spec(),
                  _vec_spec(), _vec_spec(), _vec_spec(), _full_spec()],
        out_specs=[_row_spec(), _row_spec()],
        out_shape=[
            jax.ShapeDtypeStruct((_N, _D), jnp.float32),
            jax.ShapeDtypeStruct((_N, _D), jnp.float32),
        ],
    )(sa, sb, h1, dinv, g1, beta1, b1, W2)


def _stage3_body(sa_ref, sb_ref, h2_ref, dinv_ref, g_ref, beta_ref, b_ref,
                 x_ref, o_ref):
    dinv = dinv_ref[...]
    agg = dinv * (sa_ref[...] + sb_ref[...]) + dinv * dinv * h2_ref[...] + b_ref[...]
    y = _layer_norm(agg, g_ref[...], beta_ref[...])
    o_ref[...] = jnp.maximum(y + x_ref[...], 0.0)


@jax.jit
def _stage3(sa, sb, h2, dinv, g2, beta2, b2, x):
    return pl.pallas_call(
        _stage3_body,
        grid=(_GRID,),
        in_specs=[_row_spec(), _row_spec(), _row_spec(), _col_spec(),
                  _vec_spec(), _vec_spec(), _vec_spec(), _row_spec()],
        out_specs=_row_spec(),
        out_shape=jax.ShapeDtypeStruct((_N, _D), jnp.float32),
    )(sa, sb, h2, dinv, g2, beta2, b2, x)


def kernel(x, edge_index, W1, b1, g1, beta1, W2, b2, g2, beta2):
    src = edge_index[0].astype(jnp.int32)
    dst = edge_index[1].astype(jnp.int32)

    deg_p = _deg_call(dst)
    da = deg_p[0, :_N, :1]
    db = deg_p[1, :_N, :1]

    g1r, beta1r, b1r = g1[None, :], beta1[None, :], b1[None, :]
    g2r, beta2r, b2r = g2[None, :], beta2[None, :], b2[None, :]

    h1, hs1, dinv = _stage1(x, W1, da, db)
    s1 = _agg_call(hs1, src, dst)
    h2, hs2 = _stage2(s1[0, :_N], s1[1, :_N], h1, dinv, g1r, beta1r, b1r, W2)
    s2 = _agg_call(hs2, src, dst)
    return _stage3(s2[0, :_N], s2[1, :_N], h2, dinv, g2r, beta2r, b2r, x)
